# baseline (speedup 1.0000x reference)
import functools

import jax
import jax.numpy as jnp
from jax import lax
from jax.experimental import pallas as pl
from jax.experimental.pallas import tpu as pltpu
from jax.experimental.pallas import tpu_sc as plsc

_NC = 2
_NS = 16
_CH = 64
_NBUF = 4
_NPASS = 4


def _sc_aggregate(h, srcp, dstp, n, n_pad, nchunk):
    d = h.shape[1]
    rows_zero = n_pad // _NS
    rows_out = (n // _NS) & ~7
    tail_off = rows_out * _NS
    tail = n - tail_off
    mesh = plsc.VectorSubcoreMesh(core_axis_name="c", subcore_axis_name="s",
                                  num_cores=_NC)

    def body(h_hbm, src_hbm, dst_hbm, z_hbm, out_hbm,
             idx_s, idx_d, rows, agg_sh, *gsem):
        cid = lax.axis_index("c")
        sid = lax.axis_index("s")
        w = cid * _NS + sid
        nchunk_p = nchunk // _NPASS
        pltpu.sync_copy(z_hbm, rows.at[0])
        zb = sid * rows_zero
        nfull, rem = divmod(rows_zero, _CH)
        for k in range(nfull):
            pltpu.sync_copy(rows.at[0], agg_sh.at[pl.ds(zb + k * _CH, _CH)])
        if rem:
            pltpu.sync_copy(rows.at[0].at[pl.ds(0, rem)],
                            agg_sh.at[pl.ds(zb + nfull * _CH, rem)])
        plsc.subcore_barrier()

        for p in range(_NPASS):
            pltpu.sync_copy(src_hbm.at[w, pl.ds(p * nchunk_p, nchunk_p)],
                            idx_s)
            pltpu.sync_copy(dst_hbm.at[w, pl.ds(p * nchunk_p, nchunk_p)],
                            idx_d)
            for b in range(_NBUF):
                pltpu.async_copy(h_hbm.at[idx_s.at[b]], rows.at[b], gsem[b])

            def group(g, carry):
                j0 = g * _NBUF
                for b in range(_NBUF):
                    j = j0 + b
                    pltpu.make_async_copy(
                        h_hbm.at[idx_s.at[j]], rows.at[b], gsem[b]).wait()
                    pltpu.sync_copy(rows.at[b], agg_sh.at[idx_d.at[j]],
                                    add=True)

                    @pl.when(j + _NBUF < nchunk_p)
                    def _():
                        pltpu.async_copy(
                            h_hbm.at[idx_s.at[j + _NBUF]], rows.at[b],
                            gsem[b])
                return carry

            lax.fori_loop(0, nchunk_p // _NBUF, group, 0)
        plsc.subcore_barrier()
        ob = sid * rows_out
        pltpu.sync_copy(agg_sh.at[pl.ds(ob, rows_out)],
                        out_hbm.at[cid, pl.ds(ob, rows_out)])
        if tail:
            @pl.when(sid == _NS - 1)
            def _():
                pltpu.sync_copy(agg_sh.at[pl.ds(tail_off, tail)],
                                out_hbm.at[cid, pl.ds(tail_off, tail)])

    kern = pl.kernel(
        body,
        out_type=jax.ShapeDtypeStruct((_NC, n, d), jnp.float32),
        mesh=mesh,
        scratch_types=[
            pltpu.VMEM((nchunk // _NPASS, _CH), jnp.int32),
            pltpu.VMEM((nchunk // _NPASS, _CH), jnp.int32),
            pltpu.VMEM((_NBUF, _CH, d), jnp.float32),
            pltpu.VMEM_SHARED((n_pad, d), jnp.float32),
        ] + [pltpu.SemaphoreType.DMA] * _NBUF,
    )
    return kern(h, srcp, dstp, jnp.zeros((_CH, d), jnp.float32))


def _tc_dense(h, agg2, W_msg, W_i, b_i, W_h, b_h, W1, b1, W2, b2, W3, b3):
    n, d = h.shape
    bn = 1000
    grid = n // bn
    f32 = jnp.float32

    def body(h_ref, agg_ref, wm_ref, wi_ref, bi_ref, wh_ref, bh_ref,
             w1_ref, b1_ref, w2_ref, b2_ref, w3_ref, b3_ref, out_ref):
        hb = h_ref[...]
        agg = agg_ref[0]
        for c in range(1, _NC):
            agg = agg + agg_ref[c]
        aggm = jnp.dot(agg, wm_ref[...], preferred_element_type=f32, precision=jax.lax.Precision.HIGHEST)
        gi = jnp.dot(aggm, wi_ref[...], preferred_element_type=f32, precision=jax.lax.Precision.HIGHEST) + bi_ref[...]
        gh = jnp.dot(hb, wh_ref[...], preferred_element_type=f32, precision=jax.lax.Precision.HIGHEST) + bh_ref[...]
        r = jax.nn.sigmoid(gi[:, :d] + gh[:, :d])
        z = jax.nn.sigmoid(gi[:, d:2 * d] + gh[:, d:2 * d])
        nn = jnp.tanh(gi[:, 2 * d:] + r * gh[:, 2 * d:])
        hn = (1.0 - z) * nn + z * hb
        a1 = jax.nn.relu(jnp.dot(hn, w1_ref[...], preferred_element_type=f32, precision=jax.lax.Precision.HIGHEST)
                         + b1_ref[...])
        a2 = jax.nn.relu(jnp.dot(a1, w2_ref[...], preferred_element_type=f32, precision=jax.lax.Precision.HIGHEST)
                         + b2_ref[...])
        out_ref[...] = (jnp.sum(a2 * w3_ref[...], axis=1, keepdims=True)
                        + b3_ref[...])

    h1 = W1.shape[1]
    h2 = W2.shape[1]
    full = lambda shape: pl.BlockSpec(shape, lambda i: (0,) * len(shape))
    return pl.pallas_call(
        body,
        grid=(grid,),
        in_specs=[
            pl.BlockSpec((bn, d), lambda i: (i, 0)),
            pl.BlockSpec((_NC, bn, d), lambda i: (0, i, 0)),
            full((d, d)),
            full((d, 3 * d)),
            full((1, 3 * d)),
            full((d, 3 * d)),
            full((1, 3 * d)),
            full((d, h1)),
            full((1, h1)),
            full((h1, h2)),
            full((1, h2)),
            full((1, h2)),
            full((1, 1)),
        ],
        out_specs=pl.BlockSpec((bn, 1), lambda i: (i, 0)),
        out_shape=jax.ShapeDtypeStruct((n, 1), f32),
    )(h, agg2, W_msg, W_i, b_i.reshape(1, -1), W_h, b_h.reshape(1, -1),
      W1, b1.reshape(1, -1), W2, b2.reshape(1, -1), W3.reshape(1, -1),
      b3.reshape(1, 1))


def kernel(initial_node_representation, annotations, edge_index,
           W_msg, W_i, b_i, W_h, b_h, W1, b1, W2, b2, W3, b3):
    n = initial_node_representation.shape[0]
    a = annotations.shape[1]
    d = W_msg.shape[0]
    e = edge_index.shape[0]
    h = jnp.concatenate(
        [initial_node_representation[:, : d - a], annotations], axis=1)

    nw = _NC * _NS
    epw = -(-e // nw)
    nchunk = -(-epw // _CH)
    rnd = _NBUF * _NPASS
    nchunk = -(-nchunk // rnd) * rnd
    tot = nw * nchunk * _CH
    src = edge_index[:, 0]
    dst = edge_index[:, 1]
    srcp = jnp.concatenate(
        [src, jnp.zeros((tot - e,), jnp.int32)]).reshape(nw, nchunk, _CH)
    dstp = jnp.concatenate(
        [dst, jnp.full((tot - e,), n, jnp.int32)]).reshape(nw, nchunk, _CH)

    n_pad = (n // 128 + 1) * 128
    agg2 = _sc_aggregate(h, srcp, dstp, n, n_pad, nchunk)
    return _tc_dense(h, agg2, W_msg, W_i, b_i, W_h, b_h, W1, b1, W2, b2, W3, b3)


# speedup vs baseline: 1.1323x; 1.1323x over previous
import functools

import jax
import jax.numpy as jnp
from jax import lax
from jax.experimental import pallas as pl
from jax.experimental.pallas import tpu as pltpu
from jax.experimental.pallas import tpu_sc as plsc

_NC = 2
_NS = 16
_CH = 64
_NBUF = 4
_NPASS = 4


def _sc_aggregate(h, srcp, dstp, n, n_pad, nchunk):
    d = h.shape[1]
    rows_zero = n_pad // _NS
    rows_out = (n // _NS) & ~7
    tail_off = rows_out * _NS
    tail = n - tail_off
    mesh = plsc.VectorSubcoreMesh(core_axis_name="c", subcore_axis_name="s",
                                  num_cores=_NC)

    def body(h_hbm, src_hbm, dst_hbm, z_hbm, out_hbm,
             idx_s, idx_d, rows, agg_sh, *gsem):
        cid = lax.axis_index("c")
        sid = lax.axis_index("s")
        w = cid * _NS + sid
        nchunk_p = nchunk // _NPASS
        pltpu.sync_copy(z_hbm, rows.at[0])
        zb = sid * rows_zero
        nfull, rem = divmod(rows_zero, _CH)
        for k in range(nfull):
            pltpu.sync_copy(rows.at[0], agg_sh.at[pl.ds(zb + k * _CH, _CH)])
        if rem:
            pltpu.sync_copy(rows.at[0].at[pl.ds(0, rem)],
                            agg_sh.at[pl.ds(zb + nfull * _CH, rem)])
        plsc.subcore_barrier()

        for p in range(_NPASS):
            pltpu.sync_copy(src_hbm.at[w, pl.ds(p * nchunk_p, nchunk_p)],
                            idx_s)
            pltpu.sync_copy(dst_hbm.at[w, pl.ds(p * nchunk_p, nchunk_p)],
                            idx_d)
            for b in range(_NBUF):
                pltpu.async_copy(h_hbm.at[idx_s.at[b]], rows.at[b], gsem[b])

            def group(g, carry):
                j0 = g * _NBUF
                for b in range(_NBUF):
                    j = j0 + b
                    pltpu.make_async_copy(
                        h_hbm.at[idx_s.at[j]], rows.at[b], gsem[b]).wait()
                    pltpu.sync_copy(rows.at[b], agg_sh.at[idx_d.at[j]],
                                    add=True)

                    @pl.when(j + _NBUF < nchunk_p)
                    def _():
                        pltpu.async_copy(
                            h_hbm.at[idx_s.at[j + _NBUF]], rows.at[b],
                            gsem[b])
                return carry

            lax.fori_loop(0, nchunk_p // _NBUF, group, 0)
        plsc.subcore_barrier()
        ob = sid * rows_out
        pltpu.sync_copy(agg_sh.at[pl.ds(ob, rows_out)],
                        out_hbm.at[cid, pl.ds(ob, rows_out)])
        if tail:
            @pl.when(sid == _NS - 1)
            def _():
                pltpu.sync_copy(agg_sh.at[pl.ds(tail_off, tail)],
                                out_hbm.at[cid, pl.ds(tail_off, tail)])

    kern = pl.kernel(
        body,
        out_type=jax.ShapeDtypeStruct((_NC, n, d), jnp.float32),
        mesh=mesh,
        scratch_types=[
            pltpu.VMEM((nchunk // _NPASS, _CH), jnp.int32),
            pltpu.VMEM((nchunk // _NPASS, _CH), jnp.int32),
            pltpu.VMEM((_NBUF, _CH, d), jnp.float32),
            pltpu.VMEM_SHARED((n_pad, d), jnp.float32),
        ] + [pltpu.SemaphoreType.DMA] * _NBUF,
    )
    return kern(h, srcp, dstp, jnp.zeros((_CH, d), jnp.float32))


def _tc_dense(h, agg2, W_msg, W_i, b_i, W_h, b_h, W1, b1, W2, b2, W3, b3):
    n, d = h.shape
    bn = 1000
    grid = n // bn
    f32 = jnp.float32

    def body(h_ref, agg_ref, wm_ref, wi_ref, bi_ref, wh_ref, bh_ref,
             w1_ref, b1_ref, w2_ref, b2_ref, w3_ref, b3_ref, out_ref):
        hb = h_ref[...]
        agg = agg_ref[0]
        for c in range(1, _NC):
            agg = agg + agg_ref[c]
        aggm = jnp.dot(agg, wm_ref[...], preferred_element_type=f32)
        gi = jnp.dot(aggm, wi_ref[...], preferred_element_type=f32) + bi_ref[...]
        gh = jnp.dot(hb, wh_ref[...], preferred_element_type=f32) + bh_ref[...]
        r = jax.nn.sigmoid(gi[:, :d] + gh[:, :d])
        z = jax.nn.sigmoid(gi[:, d:2 * d] + gh[:, d:2 * d])
        nn = jnp.tanh(gi[:, 2 * d:] + r * gh[:, 2 * d:])
        hn = (1.0 - z) * nn + z * hb
        a1 = jax.nn.relu(jnp.dot(hn, w1_ref[...], preferred_element_type=f32)
                         + b1_ref[...])
        a2 = jax.nn.relu(jnp.dot(a1, w2_ref[...], preferred_element_type=f32)
                         + b2_ref[...])
        out_ref[...] = (jnp.sum(a2 * w3_ref[...], axis=1, keepdims=True)
                        + b3_ref[...])

    h1 = W1.shape[1]
    h2 = W2.shape[1]
    full = lambda shape: pl.BlockSpec(shape, lambda i: (0,) * len(shape))
    return pl.pallas_call(
        body,
        grid=(grid,),
        in_specs=[
            pl.BlockSpec((bn, d), lambda i: (i, 0)),
            pl.BlockSpec((_NC, bn, d), lambda i: (0, i, 0)),
            full((d, d)),
            full((d, 3 * d)),
            full((1, 3 * d)),
            full((d, 3 * d)),
            full((1, 3 * d)),
            full((d, h1)),
            full((1, h1)),
            full((h1, h2)),
            full((1, h2)),
            full((1, h2)),
            full((1, 1)),
        ],
        out_specs=pl.BlockSpec((bn, 1), lambda i: (i, 0)),
        out_shape=jax.ShapeDtypeStruct((n, 1), f32),
    )(h, agg2, W_msg, W_i, b_i.reshape(1, -1), W_h, b_h.reshape(1, -1),
      W1, b1.reshape(1, -1), W2, b2.reshape(1, -1), W3.reshape(1, -1),
      b3.reshape(1, 1))


def kernel(initial_node_representation, annotations, edge_index,
           W_msg, W_i, b_i, W_h, b_h, W1, b1, W2, b2, W3, b3):
    n = initial_node_representation.shape[0]
    a = annotations.shape[1]
    d = W_msg.shape[0]
    e = edge_index.shape[0]
    h = jnp.concatenate(
        [initial_node_representation[:, : d - a], annotations], axis=1)

    nw = _NC * _NS
    epw = -(-e // nw)
    nchunk = -(-epw // _CH)
    rnd = _NBUF * _NPASS
    nchunk = -(-nchunk // rnd) * rnd
    tot = nw * nchunk * _CH
    src = edge_index[:, 0]
    dst = edge_index[:, 1]
    srcp = jnp.concatenate(
        [src, jnp.zeros((tot - e,), jnp.int32)]).reshape(nw, nchunk, _CH)
    dstp = jnp.concatenate(
        [dst, jnp.full((tot - e,), n, jnp.int32)]).reshape(nw, nchunk, _CH)

    n_pad = (n // 128 + 1) * 128
    agg2 = _sc_aggregate(h, srcp, dstp, n, n_pad, nchunk)
    return _tc_dense(h, agg2, W_msg, W_i, b_i, W_h, b_h, W1, b1, W2, b2, W3, b3)


# speedup vs baseline: 1.1417x; 1.0082x over previous
import functools

import jax
import jax.numpy as jnp
from jax import lax
from jax.experimental import pallas as pl
from jax.experimental.pallas import tpu as pltpu
from jax.experimental.pallas import tpu_sc as plsc

_NC = 2
_NS = 16
_CH = 64
_NBUF = 4
_NPASS = 4


def _sc_aggregate(h, srcp, dstp, n, n_pad, nchunk):
    d = h.shape[1]
    rows_zero = n_pad // _NS
    rows_out = (n // _NS) & ~7
    tail_off = rows_out * _NS
    tail = n - tail_off
    mesh = plsc.VectorSubcoreMesh(core_axis_name="c", subcore_axis_name="s",
                                  num_cores=_NC)

    def body(h_hbm, src_hbm, dst_hbm, z_hbm, out_hbm,
             idx_s, idx_d, rows, agg_sh, *gsem):
        cid = lax.axis_index("c")
        sid = lax.axis_index("s")
        w = cid * _NS + sid
        nchunk_p = nchunk // _NPASS
        pltpu.sync_copy(z_hbm, rows.at[0])
        zb = sid * rows_zero
        nfull, rem = divmod(rows_zero, _CH)
        for k in range(nfull):
            pltpu.sync_copy(rows.at[0], agg_sh.at[pl.ds(zb + k * _CH, _CH)])
        if rem:
            pltpu.sync_copy(rows.at[0].at[pl.ds(0, rem)],
                            agg_sh.at[pl.ds(zb + nfull * _CH, rem)])
        plsc.subcore_barrier()

        for p in range(_NPASS):
            pltpu.sync_copy(src_hbm.at[w, pl.ds(p * nchunk_p, nchunk_p)],
                            idx_s)
            pltpu.sync_copy(dst_hbm.at[w, pl.ds(p * nchunk_p, nchunk_p)],
                            idx_d)
            for b in range(_NBUF):
                pltpu.async_copy(h_hbm.at[idx_s.at[b]], rows.at[b], gsem[b])

            def group(g, carry):
                j0 = g * _NBUF
                for b in range(_NBUF):
                    j = j0 + b
                    pltpu.make_async_copy(
                        h_hbm.at[idx_s.at[j]], rows.at[b], gsem[b]).wait()
                    pltpu.sync_copy(rows.at[b], agg_sh.at[idx_d.at[j]],
                                    add=True)

                    @pl.when(j + _NBUF < nchunk_p)
                    def _():
                        pltpu.async_copy(
                            h_hbm.at[idx_s.at[j + _NBUF]], rows.at[b],
                            gsem[b])
                return carry

            lax.fori_loop(0, nchunk_p // _NBUF, group, 0)
        plsc.subcore_barrier()
        ob = sid * rows_out
        pltpu.sync_copy(agg_sh.at[pl.ds(ob, rows_out)],
                        out_hbm.at[cid, pl.ds(ob, rows_out)])
        if tail:
            @pl.when(sid == _NS - 1)
            def _():
                pltpu.sync_copy(agg_sh.at[pl.ds(tail_off, tail)],
                                out_hbm.at[cid, pl.ds(tail_off, tail)])

    kern = pl.kernel(
        body,
        out_type=jax.ShapeDtypeStruct((_NC, n, d), jnp.float32),
        mesh=mesh,
        scratch_types=[
            pltpu.VMEM((nchunk // _NPASS, _CH), jnp.int32),
            pltpu.VMEM((nchunk // _NPASS, _CH), jnp.int32),
            pltpu.VMEM((_NBUF, _CH, d), jnp.float32),
            pltpu.VMEM_SHARED((n_pad, d), jnp.float32),
        ] + [pltpu.SemaphoreType.DMA] * _NBUF,
    )
    return kern(h, srcp, dstp, jnp.zeros((_CH, d), jnp.float32))


def _tc_dense(h, agg2, W_msg, W_i, b_i, W_h, b_h, W1, b1, W2, b2, W3, b3):
    n, d = h.shape
    bn = 2000
    grid = n // bn
    f32 = jnp.float32

    def body(h_ref, agg_ref, wm_ref, wi_ref, bi_ref, wh_ref, bh_ref,
             w1_ref, b1_ref, w2_ref, b2_ref, w3_ref, b3_ref, out_ref):
        hb = h_ref[...]
        agg = agg_ref[0]
        for c in range(1, _NC):
            agg = agg + agg_ref[c]
        aggm = jnp.dot(agg, wm_ref[...], preferred_element_type=f32)
        gi = jnp.dot(aggm, wi_ref[...], preferred_element_type=f32) + bi_ref[...]
        gh = jnp.dot(hb, wh_ref[...], preferred_element_type=f32) + bh_ref[...]
        r = jax.nn.sigmoid(gi[:, :d] + gh[:, :d])
        z = jax.nn.sigmoid(gi[:, d:2 * d] + gh[:, d:2 * d])
        nn = jnp.tanh(gi[:, 2 * d:] + r * gh[:, 2 * d:])
        hn = (1.0 - z) * nn + z * hb
        a1 = jax.nn.relu(jnp.dot(hn, w1_ref[...], preferred_element_type=f32)
                         + b1_ref[...])
        a2 = jax.nn.relu(jnp.dot(a1, w2_ref[...], preferred_element_type=f32)
                         + b2_ref[...])
        out_ref[...] = (jnp.sum(a2 * w3_ref[...], axis=1, keepdims=True)
                        + b3_ref[...])

    h1 = W1.shape[1]
    h2 = W2.shape[1]
    full = lambda shape: pl.BlockSpec(shape, lambda i: (0,) * len(shape))
    return pl.pallas_call(
        body,
        grid=(grid,),
        in_specs=[
            pl.BlockSpec((bn, d), lambda i: (i, 0)),
            pl.BlockSpec((_NC, bn, d), lambda i: (0, i, 0)),
            full((d, d)),
            full((d, 3 * d)),
            full((1, 3 * d)),
            full((d, 3 * d)),
            full((1, 3 * d)),
            full((d, h1)),
            full((1, h1)),
            full((h1, h2)),
            full((1, h2)),
            full((1, h2)),
            full((1, 1)),
        ],
        out_specs=pl.BlockSpec((bn, 1), lambda i: (i, 0)),
        out_shape=jax.ShapeDtypeStruct((n, 1), f32),
    )(h, agg2, W_msg, W_i, b_i.reshape(1, -1), W_h, b_h.reshape(1, -1),
      W1, b1.reshape(1, -1), W2, b2.reshape(1, -1), W3.reshape(1, -1),
      b3.reshape(1, 1))


def kernel(initial_node_representation, annotations, edge_index,
           W_msg, W_i, b_i, W_h, b_h, W1, b1, W2, b2, W3, b3):
    n = initial_node_representation.shape[0]
    a = annotations.shape[1]
    d = W_msg.shape[0]
    e = edge_index.shape[0]
    h = jnp.concatenate(
        [initial_node_representation[:, : d - a], annotations], axis=1)

    nw = _NC * _NS
    epw = -(-e // nw)
    nchunk = -(-epw // _CH)
    rnd = _NBUF * _NPASS
    nchunk = -(-nchunk // rnd) * rnd
    tot = nw * nchunk * _CH
    src = edge_index[:, 0]
    dst = edge_index[:, 1]
    srcp = jnp.concatenate(
        [src, jnp.zeros((tot - e,), jnp.int32)]).reshape(nw, nchunk, _CH)
    dstp = jnp.concatenate(
        [dst, jnp.full((tot - e,), n, jnp.int32)]).reshape(nw, nchunk, _CH)

    n_pad = (n // 128 + 1) * 128
    agg2 = _sc_aggregate(h, srcp, dstp, n, n_pad, nchunk)
    return _tc_dense(h, agg2, W_msg, W_i, b_i, W_h, b_h, W1, b1, W2, b2, W3, b3)
